# baseline (speedup 1.0000x reference)
import jax
import jax.numpy as jnp
from jax import lax
from jax.experimental import pallas as pl
from jax.experimental.pallas import tpu as pltpu
from jax.experimental.pallas import tpu_sc as plsc

BATCH = 16384
HIST = 50
DIM = 64
NB = 128
NBLK = HIST * (BATCH // NB)
NC = 2
NS = 16
NW = NC * NS
BLK_PER_W = NBLK // NW
L = 16
TPAIR = 500000


def _body(xt_hbm, tab_hbm, out_hbm, idx_v, pidx_v, par_v, rows_v, t_v,
          gsem, wsem):
    wid = lax.axis_index("s") * NC + lax.axis_index("c")
    g0 = wid * BLK_PER_W

    pltpu.sync_copy(xt_hbm.at[pl.ds(g0, BLK_PER_W)], idx_v)

    iota = lax.iota(jnp.int32, L)
    rowidx = [iota + cc * L for cc in range(NB // L)]

    def prep_and_fire(g, s):
        for cc in range(NB // L):
            v = idx_v[g, pl.ds(cc * L, L)]
            pidx_v[s, pl.ds(cc * L, L)] = lax.shift_right_logical(v, 1)
            par_v[s, pl.ds(cc * L, L)] = lax.shift_left(v & 1, 6)
        pltpu.async_copy(tab_hbm.at[pidx_v.at[s]], rows_v.at[s], gsem.at[s])

    prep_and_fire(0, 0)

    @pl.loop(0, BLK_PER_W, step=2)
    def _blk(gl):
        for s in range(2):
            g = gl + s

            @pl.when(g < BLK_PER_W - 1)
            def _():
                prep_and_fire(g + 1, 1 - s)

            pltpu.make_async_copy(
                tab_hbm.at[pl.ds(0, NB)], rows_v.at[s], gsem.at[s]
            ).wait()

            @pl.when(g >= 2)
            def _():
                for tr in range(8):
                    pltpu.make_async_copy(
                        t_v.at[s].at[pl.ds(tr * 8, 8)],
                        out_hbm.at[0, tr, 0],
                        wsem.at[s],
                    ).wait()

            colbase = [par_v[s, pl.ds(cc * L, L)] for cc in range(NB // L)]

            @pl.loop(0, DIM)
            def _j(j):
                for cc in range(NB // L):
                    vals = plsc.load_gather(
                        rows_v.at[s], [rowidx[cc], colbase[cc] + j]
                    )
                    t_v[s, j, pl.ds(cc * L, L)] = vals

            gg = g0 + g
            h = gg // 128
            bc = gg % 128
            for tr in range(8):
                pltpu.async_copy(
                    t_v.at[s].at[pl.ds(tr * 8, 8)],
                    out_hbm.at[h, tr, bc],
                    wsem.at[s],
                )

    for s in range(2):
        for tr in range(8):
            pltpu.make_async_copy(
                t_v.at[s].at[pl.ds(tr * 8, 8)],
                out_hbm.at[0, tr, 0],
                wsem.at[s],
            ).wait()


def kernel(x, table):
    xt = x.T.reshape(NBLK, NB).astype(jnp.int32)
    tab2 = table.reshape(TPAIR, NB)
    mesh = plsc.VectorSubcoreMesh(core_axis_name="c", subcore_axis_name="s")
    grab = pl.kernel(
        _body,
        out_type=jax.ShapeDtypeStruct((HIST, 8, 128, 8, NB), jnp.float32),
        mesh=mesh,
        scratch_types=[
            pltpu.VMEM((BLK_PER_W, NB), jnp.int32),
            pltpu.VMEM((2, NB), jnp.int32),
            pltpu.VMEM((2, NB), jnp.int32),
            pltpu.VMEM((2, NB, NB), jnp.float32),
            pltpu.VMEM((2, DIM, NB), jnp.float32),
            pltpu.SemaphoreType.DMA((2,)),
            pltpu.SemaphoreType.DMA((2,)),
        ],
        compiler_params=pltpu.CompilerParams(
            use_tc_tiling_on_sc=False, needs_layout_passes=False
        ),
    )
    out5 = grab(xt, tab2)
    return jnp.transpose(out5, (2, 4, 0, 1, 3)).reshape(BATCH, HIST, DIM)


# speedup vs baseline: 1.8003x; 1.8003x over previous
import jax
import jax.numpy as jnp
from jax import lax
from jax.experimental import pallas as pl
from jax.experimental.pallas import tpu as pltpu
from jax.experimental.pallas import tpu_sc as plsc

BATCH = 16384
HIST = 50
DIM = 64
NB = 128
NBLK = HIST * (BATCH // NB)
NC = 2
NS = 16
NW = NC * NS
BLK_PER_W = NBLK // NW
L = 16
TSTRIDE = NB + 1


def _body(xt_hbm, tab_hbm, out_hbm, idx_v, rows_v, t_v, gsem, wsem):
    wid = lax.axis_index("s") * NC + lax.axis_index("c")
    g0 = wid * BLK_PER_W

    pltpu.sync_copy(xt_hbm.at[pl.ds(g0, BLK_PER_W)], idx_v)

    iota = lax.iota(jnp.int32, L)
    jidx = [iota + k * L for k in range(DIM // L)]

    def fire(g, s):
        pltpu.async_copy(
            tab_hbm.at[idx_v.at[g]], rows_v.at[s], gsem.at[s]
        )

    fire(0, 0)

    @pl.loop(0, BLK_PER_W, step=2)
    def _blk(gl):
        for s in range(2):
            g = gl + s

            @pl.when(g < BLK_PER_W - 1)
            def _():
                fire(g + 1, 1 - s)

            pltpu.make_async_copy(
                tab_hbm.at[pl.ds(0, NB)], rows_v.at[s], gsem.at[s]
            ).wait()

            @pl.when(g >= 2)
            def _():
                for tr in range(8):
                    pltpu.make_async_copy(
                        t_v.at[s].at[pl.ds(tr * 8, 8), pl.ds(0, NB)],
                        out_hbm.at[0, tr, 0],
                        wsem.at[s],
                    ).wait()

            @pl.loop(0, NB)
            def _c(c):
                cidx = jnp.full((L,), c, jnp.int32)
                for k in range(DIM // L):
                    vals = rows_v[s, c, pl.ds(k * L, L)]
                    plsc.store_scatter(t_v.at[s], [jidx[k], cidx], vals)

            gg = g0 + g
            h = gg // 128
            bc = gg % 128
            for tr in range(8):
                pltpu.async_copy(
                    t_v.at[s].at[pl.ds(tr * 8, 8), pl.ds(0, NB)],
                    out_hbm.at[h, tr, bc],
                    wsem.at[s],
                )

    for s in range(2):
        for tr in range(8):
            pltpu.make_async_copy(
                t_v.at[s].at[pl.ds(tr * 8, 8), pl.ds(0, NB)],
                out_hbm.at[0, tr, 0],
                wsem.at[s],
            ).wait()


def kernel(x, table):
    xt = x.T.reshape(NBLK, NB).astype(jnp.int32)
    mesh = plsc.VectorSubcoreMesh(core_axis_name="c", subcore_axis_name="s")
    grab = pl.kernel(
        _body,
        out_type=jax.ShapeDtypeStruct((HIST, 8, 128, 8, NB), jnp.float32),
        mesh=mesh,
        scratch_types=[
            pltpu.VMEM((BLK_PER_W, NB), jnp.int32),
            pltpu.VMEM((2, NB, DIM), jnp.float32),
            pltpu.VMEM((2, DIM, TSTRIDE), jnp.float32),
            pltpu.SemaphoreType.DMA((2,)),
            pltpu.SemaphoreType.DMA((2,)),
        ],
        compiler_params=pltpu.CompilerParams(
            use_tc_tiling_on_sc=False, needs_layout_passes=False
        ),
    )
    out5 = grab(xt, table)
    return jnp.transpose(out5, (2, 4, 0, 1, 3)).reshape(BATCH, HIST, DIM)


# speedup vs baseline: 1.8464x; 1.0256x over previous
import jax
import jax.numpy as jnp
from jax import lax
from jax.experimental import pallas as pl
from jax.experimental.pallas import tpu as pltpu
from jax.experimental.pallas import tpu_sc as plsc

BATCH = 16384
HIST = 50
DIM = 64
NB = 128
NBLK = HIST * (BATCH // NB)
NC = 2
NS = 16
NW = NC * NS
BLK_PER_W = NBLK // NW
L = 16
TSTRIDE = NB + 1


def _body(xt_hbm, tab_hbm, out_hbm, idx_v, rows_v, t_v, gsem, wsem):
    wid = lax.axis_index("s") * NC + lax.axis_index("c")
    g0 = wid * BLK_PER_W

    pltpu.sync_copy(xt_hbm.at[pl.ds(g0, BLK_PER_W)], idx_v)

    iota = lax.iota(jnp.int32, L)
    jtr = [(iota + k * L) // 8 for k in range(DIM // L)]
    jr = [(iota + k * L) % 8 for k in range(DIM // L)]

    def fire(g, s):
        pltpu.async_copy(tab_hbm.at[idx_v.at[g]], rows_v.at[s], gsem.at[s])

    def drain_writes(s):
        pltpu.make_async_copy(
            tab_hbm.at[pl.ds(0, NB)], rows_v.at[s], wsem.at[s]
        ).wait()

    fire(0, 0)

    @pl.loop(0, BLK_PER_W, step=2)
    def _blk(gl):
        for s in range(2):
            g = gl + s

            @pl.when(g < BLK_PER_W - 1)
            def _():
                fire(g + 1, 1 - s)

            pltpu.make_async_copy(
                tab_hbm.at[pl.ds(0, NB)], rows_v.at[s], gsem.at[s]
            ).wait()

            @pl.when(g >= 2)
            def _():
                drain_writes(s)

            @pl.loop(0, NB, unroll=4)
            def _c(c):
                cidx = jnp.full((L,), c, jnp.int32)
                for k in range(DIM // L):
                    vals = rows_v[s, c, pl.ds(k * L, L)]
                    plsc.store_scatter(t_v.at[s], [jtr[k], jr[k], cidx], vals)

            gg = g0 + g
            h = gg // 128
            bc = gg % 128
            pltpu.async_copy(
                t_v.at[s].at[:, :, pl.ds(0, NB)],
                out_hbm.at[h, :, bc],
                wsem.at[s],
            )

    for s in range(2):
        drain_writes(s)


def kernel(x, table):
    xt = x.T.reshape(NBLK, NB).astype(jnp.int32)
    mesh = plsc.VectorSubcoreMesh(core_axis_name="c", subcore_axis_name="s")
    grab = pl.kernel(
        _body,
        out_type=jax.ShapeDtypeStruct((HIST, 8, 128, 8, NB), jnp.float32),
        mesh=mesh,
        scratch_types=[
            pltpu.VMEM((BLK_PER_W, NB), jnp.int32),
            pltpu.VMEM((2, NB, DIM), jnp.float32),
            pltpu.VMEM((2, 8, 8, TSTRIDE), jnp.float32),
            pltpu.SemaphoreType.DMA((2,)),
            pltpu.SemaphoreType.DMA((2,)),
        ],
        compiler_params=pltpu.CompilerParams(
            use_tc_tiling_on_sc=False, needs_layout_passes=False
        ),
    )
    out5 = grab(xt, table)
    return jnp.transpose(out5, (2, 4, 0, 1, 3)).reshape(BATCH, HIST, DIM)
